# Initial kernel scaffold; baseline (speedup 1.0000x reference)
import jax
import jax.numpy as jnp
from jax.experimental import pallas as pl


def kernel(x, Wq, Wk, Wv, Wo, betas, W1, b1, W2, b2, ln_g, ln_b, Ws, bs):
    raise NotImplementedError("write your pallas kernel here")


# speedup vs baseline: 1.1460x; 1.1460x over previous
import jax
import jax.numpy as jnp
from jax import lax
from jax.experimental import pallas as pl
from jax.experimental.pallas import tpu as pltpu

B = 2
S = 4096
D = 1024
DH = 4096
DK = 64
DV = 64
H = 16
FULL = 2048
SEG = 256
G = B * (S // FULL)
NTOK = SEG * (S // FULL)


def _elu1(t):
    return jnp.where(t > 0, t + 1.0, jnp.exp(t))


def _routing_body(x_ref, ws_ref, bs_ref, s_ref, mask_ref, p_ref, xsel_ref):
    xseg = x_ref[0]
    ws = ws_ref[...]
    s = jnp.dot(xseg, ws, preferred_element_type=jnp.float32) + bs_ref[0, 0]
    s_ref[0] = s

    u = lax.bitcast_convert_type(s, jnp.uint32)
    u = jnp.where(u >= jnp.uint32(0x80000000), ~u, u | jnp.uint32(0x80000000))

    def bit_step(i, t):
        cand = t | lax.shift_left(jnp.uint32(1), jnp.uint32(31) - i.astype(jnp.uint32))
        cnt = jnp.sum((u >= cand).astype(jnp.int32))
        return jnp.where(cnt >= SEG, cand, t)

    thr = lax.fori_loop(0, 32, bit_step, jnp.uint32(0))

    gtf = (u > thr).astype(jnp.float32)
    eqf = (u == thr).astype(jnp.float32)
    ri = lax.broadcasted_iota(jnp.int32, (FULL, FULL), 0)
    ci = lax.broadcasted_iota(jnp.int32, (FULL, FULL), 1)
    lf = (ci < ri).astype(jnp.float32)
    need = jnp.float32(SEG) - jnp.sum(gtf)
    erank = jnp.dot(lf, eqf, preferred_element_type=jnp.float32)
    m = gtf + eqf * (erank < need).astype(jnp.float32)
    mask_ref[0] = m
    rank = jnp.dot(lf, m, preferred_element_type=jnp.float32)
    cols = lax.broadcasted_iota(jnp.int32, (FULL, SEG), 1).astype(jnp.float32)
    p = m * (rank == cols).astype(jnp.float32)
    p_ref[0] = p
    xsel_ref[0] = lax.dot_general(p, xseg, (((0,), (0,)), ((), ())),
                                  preferred_element_type=jnp.float32)


def _attn_body(xsel_ref, wq_ref, wk_ref, wv_ref, wo_ref, beta_ref, out_ref):
    xs = xsel_ref[0]
    acc = jnp.zeros((NTOK, D), jnp.float32)
    for h in range(H):
        q = jnp.dot(xs, wq_ref[h], preferred_element_type=jnp.float32)
        k = jnp.dot(xs, wk_ref[h], preferred_element_type=jnp.float32)
        v = jnp.dot(xs, wv_ref[h], preferred_element_type=jnp.float32)
        g = 1.0 / (1.0 + jnp.exp(-beta_ref[h]))
        mem = jnp.zeros((DK, DV), jnp.float32)
        zrow = jnp.ones((1, DK), jnp.float32) / DK
        outs = []
        for sgi in range(NTOK // SEG):
            qs = q[sgi * SEG:(sgi + 1) * SEG]
            ks = k[sgi * SEG:(sgi + 1) * SEG]
            vs = v[sgi * SEG:(sgi + 1) * SEG]
            sq = _elu1(qs)
            att_mem = jnp.dot(sq, mem, preferred_element_type=jnp.float32)
            att_mem = att_mem / lax.dot_general(sq, zrow, (((1,), (1,)), ((), ())),
                                                preferred_element_type=jnp.float32)
            scores = lax.dot_general(qs, ks, (((1,), (1,)), ((), ())),
                                     preferred_element_type=jnp.float32) / 8.0
            att_dot = jnp.dot(jax.nn.softmax(scores, axis=-1), vs,
                              preferred_element_type=jnp.float32)
            sk = _elu1(ks)
            mem = mem + lax.dot_general(sk, vs, (((0,), (0,)), ((), ())),
                                        preferred_element_type=jnp.float32)
            zrow = zrow + jnp.sum(sk, axis=0, keepdims=True)
            outs.append(g * att_mem + (1.0 - g) * att_dot)
        att_h = jnp.concatenate(outs, axis=0)
        acc = acc + jnp.dot(att_h, wo_ref[h], preferred_element_type=jnp.float32)
    out_ref[0] = acc


def _mlp_body(h_ref, w1_ref, b1_ref, w2_ref, b2_ref, out_ref):
    t = h_ref[...]
    a = jnp.dot(t, w1_ref[...], preferred_element_type=jnp.float32) + b1_ref[...]
    a = jnp.maximum(a, 0.0)
    out_ref[...] = jnp.dot(a, w2_ref[...], preferred_element_type=jnp.float32) + b2_ref[...]


def _ln_body(x_ref, p_ref, h_ref, g_ref, b_ref, out_ref):
    xseg = x_ref[0]
    xup = xseg + jnp.dot(p_ref[0], h_ref[0], preferred_element_type=jnp.float32)
    mu = jnp.mean(xup, axis=1, keepdims=True)
    var = jnp.mean((xup - mu) ** 2, axis=1, keepdims=True)
    out_ref[0] = (xup - mu) / jnp.sqrt(var + 1e-5) * g_ref[...] + b_ref[...]


def kernel(x, Wq, Wk, Wv, Wo, betas, W1, b1, W2, b2, ln_g, ln_b, Ws, bs):
    f32 = jnp.float32
    x4 = x.reshape(G, FULL, D)

    s4, mask4, p4, xsel4 = pl.pallas_call(
        _routing_body,
        grid=(G,),
        in_specs=[
            pl.BlockSpec((1, FULL, D), lambda i: (i, 0, 0)),
            pl.BlockSpec((D, 1), lambda i: (0, 0)),
            pl.BlockSpec(memory_space=pltpu.SMEM),
        ],
        out_specs=[
            pl.BlockSpec((1, FULL, 1), lambda i: (i, 0, 0)),
            pl.BlockSpec((1, FULL, 1), lambda i: (i, 0, 0)),
            pl.BlockSpec((1, FULL, SEG), lambda i: (i, 0, 0)),
            pl.BlockSpec((1, SEG, D), lambda i: (i, 0, 0)),
        ],
        out_shape=[
            jax.ShapeDtypeStruct((G, FULL, 1), f32),
            jax.ShapeDtypeStruct((G, FULL, 1), f32),
            jax.ShapeDtypeStruct((G, FULL, SEG), f32),
            jax.ShapeDtypeStruct((G, SEG, D), f32),
        ],
    )(x4, Ws, bs.reshape(1, 1))

    xsel = xsel4.reshape(B, NTOK, D)
    wq_r = Wq.reshape(D, H, DK).transpose(1, 0, 2)
    wk_r = Wk.reshape(D, H, DK).transpose(1, 0, 2)
    wv_r = Wv.reshape(D, H, DV).transpose(1, 0, 2)
    wo_r = Wo.reshape(H, DV, D)
    beta_r = betas.reshape(H, 1, DV)

    h_att = pl.pallas_call(
        _attn_body,
        grid=(B,),
        in_specs=[
            pl.BlockSpec((1, NTOK, D), lambda i: (i, 0, 0)),
            pl.BlockSpec((H, D, DK), lambda i: (0, 0, 0)),
            pl.BlockSpec((H, D, DK), lambda i: (0, 0, 0)),
            pl.BlockSpec((H, D, DV), lambda i: (0, 0, 0)),
            pl.BlockSpec((H, DV, D), lambda i: (0, 0, 0)),
            pl.BlockSpec((H, 1, DV), lambda i: (0, 0, 0)),
        ],
        out_specs=pl.BlockSpec((1, NTOK, D), lambda i: (i, 0, 0)),
        out_shape=jax.ShapeDtypeStruct((B, NTOK, D), f32),
    )(xsel, wq_r, wk_r, wv_r, wo_r, beta_r)

    tok = B * NTOK
    blk = 256
    h_mlp = pl.pallas_call(
        _mlp_body,
        grid=(tok // blk,),
        in_specs=[
            pl.BlockSpec((blk, D), lambda i: (i, 0)),
            pl.BlockSpec((D, DH), lambda i: (0, 0)),
            pl.BlockSpec((1, DH), lambda i: (0, 0)),
            pl.BlockSpec((DH, D), lambda i: (0, 0)),
            pl.BlockSpec((1, D), lambda i: (0, 0)),
        ],
        out_specs=pl.BlockSpec((blk, D), lambda i: (i, 0)),
        out_shape=jax.ShapeDtypeStruct((tok, D), f32),
    )(h_att.reshape(tok, D), W1, b1.reshape(1, DH), W2, b2.reshape(1, D))

    out4 = pl.pallas_call(
        _ln_body,
        grid=(G,),
        in_specs=[
            pl.BlockSpec((1, FULL, D), lambda i: (i, 0, 0)),
            pl.BlockSpec((1, FULL, SEG), lambda i: (i, 0, 0)),
            pl.BlockSpec((1, SEG, D), lambda i: (i, 0, 0)),
            pl.BlockSpec((1, D), lambda i: (0, 0)),
            pl.BlockSpec((1, D), lambda i: (0, 0)),
        ],
        out_specs=pl.BlockSpec((1, FULL, D), lambda i: (i, 0, 0)),
        out_shape=jax.ShapeDtypeStruct((G, FULL, D), f32),
    )(x4, p4, h_mlp.reshape(G, SEG, D), ln_g.reshape(1, D), ln_b.reshape(1, D))

    out = out4.reshape(B, S, D)
    return out, mask4.reshape(B * S, 1), s4.reshape(B * S, 1)


# speedup vs baseline: 1.6908x; 1.4754x over previous
import jax
import jax.numpy as jnp
from jax import lax
from jax.experimental import pallas as pl
from jax.experimental.pallas import tpu as pltpu

B = 2
S = 4096
D = 1024
DH = 4096
DK = 64
DV = 64
H = 16
FULL = 2048
SEG = 256
G = B * (S // FULL)
NTOK = SEG * (S // FULL)


def _elu1(t):
    return jnp.where(t > 0, t + 1.0, jnp.exp(t))


_ROWS = 16
_LANES = FULL // _ROWS


def _routing_body(x_ref, ws_ref, bs_ref, s_ref, mask_ref, p_ref, xsel_ref):
    xseg = x_ref[0]
    ws = ws_ref[...]
    s = jnp.dot(xseg, ws, preferred_element_type=jnp.float32) + bs_ref[0, 0]
    s_ref[0] = s

    s2 = s.reshape(_ROWS, _LANES)
    u = lax.bitcast_convert_type(s2, jnp.uint32)
    u = jnp.where(u >= jnp.uint32(0x80000000), ~u, u | jnp.uint32(0x80000000))

    def bit_step(i, t):
        cand = t | lax.shift_left(jnp.uint32(1), jnp.uint32(31) - i.astype(jnp.uint32))
        cnt = jnp.sum((u >= cand).astype(jnp.int32))
        return jnp.where(cnt >= SEG, cand, t)

    thr = lax.fori_loop(0, 32, bit_step, jnp.uint32(0))

    gtf = (u > thr).astype(jnp.float32)
    eqf = (u == thr).astype(jnp.float32)
    need = jnp.float32(SEG) - jnp.sum(gtf)
    cu = lax.broadcasted_iota(jnp.int32, (_LANES, _LANES), 0)
    cv = lax.broadcasted_iota(jnp.int32, (_LANES, _LANES), 1)
    c128 = (cu < cv).astype(jnp.float32)
    ru = lax.broadcasted_iota(jnp.int32, (_ROWS, _ROWS), 0)
    rv = lax.broadcasted_iota(jnp.int32, (_ROWS, _ROWS), 1)
    r16 = (ru < rv).astype(jnp.float32)
    eprefix = jnp.dot(eqf, c128, preferred_element_type=jnp.float32)
    gprefix = jnp.dot(gtf, c128, preferred_element_type=jnp.float32)
    esum = jnp.sum(eqf, axis=1, keepdims=True)
    gsum = jnp.sum(gtf, axis=1, keepdims=True)
    eoff = lax.dot_general(r16, esum, (((0,), (0,)), ((), ())),
                           preferred_element_type=jnp.float32)
    goff = lax.dot_general(r16, gsum, (((0,), (0,)), ((), ())),
                           preferred_element_type=jnp.float32)
    erank = eprefix + eoff
    grank = gprefix + goff
    m2 = gtf + eqf * (erank < need).astype(jnp.float32)
    mask_ref[0] = m2
    rank = grank + jnp.minimum(erank, need)
    e_rows = (lax.broadcasted_iota(jnp.int32, (FULL, _ROWS), 0) // _LANES
              == lax.broadcasted_iota(jnp.int32, (FULL, _ROWS), 1)).astype(jnp.float32)
    lane_sel = (lax.broadcasted_iota(jnp.int32, (FULL, _LANES), 0) % _LANES
                == lax.broadcasted_iota(jnp.int32, (FULL, _LANES), 1))
    rank_rows = jnp.dot(e_rows, rank, preferred_element_type=jnp.float32)
    m_rows = jnp.dot(e_rows, m2, preferred_element_type=jnp.float32)
    rank_col = jnp.sum(jnp.where(lane_sel, rank_rows, 0.0), axis=1, keepdims=True)
    m_col = jnp.sum(jnp.where(lane_sel, m_rows, 0.0), axis=1, keepdims=True)
    cols = lax.broadcasted_iota(jnp.int32, (FULL, SEG), 1).astype(jnp.float32)
    p = m_col * (rank_col == cols).astype(jnp.float32)
    p_ref[0] = p
    xsel_ref[0] = lax.dot_general(p, xseg, (((0,), (0,)), ((), ())),
                                  preferred_element_type=jnp.float32)


def _attn_body(xsel_ref, wq_ref, wk_ref, wv_ref, wo_ref, beta_ref, out_ref):
    xs = xsel_ref[0]
    q_all = jnp.dot(xs, wq_ref[...], preferred_element_type=jnp.float32)
    k_all = jnp.dot(xs, wk_ref[...], preferred_element_type=jnp.float32)
    v_all = jnp.dot(xs, wv_ref[...], preferred_element_type=jnp.float32)
    heads = []
    for h in range(H):
        q = q_all[:, h * DK:(h + 1) * DK]
        k = k_all[:, h * DK:(h + 1) * DK]
        v = v_all[:, h * DV:(h + 1) * DV]
        g = 1.0 / (1.0 + jnp.exp(-beta_ref[h]))
        mem = jnp.zeros((DK, DV), jnp.float32)
        zrow = jnp.ones((1, DK), jnp.float32) / DK
        outs = []
        for sgi in range(NTOK // SEG):
            qs = q[sgi * SEG:(sgi + 1) * SEG]
            ks = k[sgi * SEG:(sgi + 1) * SEG]
            vs = v[sgi * SEG:(sgi + 1) * SEG]
            sq = _elu1(qs)
            att_mem = jnp.dot(sq, mem, preferred_element_type=jnp.float32)
            att_mem = att_mem / lax.dot_general(sq, zrow, (((1,), (1,)), ((), ())),
                                                preferred_element_type=jnp.float32)
            scores = lax.dot_general(qs, ks, (((1,), (1,)), ((), ())),
                                     preferred_element_type=jnp.float32) / 8.0
            att_dot = jnp.dot(jax.nn.softmax(scores, axis=-1), vs,
                              preferred_element_type=jnp.float32)
            sk = _elu1(ks)
            mem = mem + lax.dot_general(sk, vs, (((0,), (0,)), ((), ())),
                                        preferred_element_type=jnp.float32)
            zrow = zrow + jnp.sum(sk, axis=0, keepdims=True)
            outs.append(g * att_mem + (1.0 - g) * att_dot)
        heads.append(jnp.concatenate(outs, axis=0))
    att_full = jnp.concatenate(heads, axis=1)
    out_ref[0] = jnp.dot(att_full, wo_ref[...], preferred_element_type=jnp.float32)


def _mlp_body(h_ref, w1_ref, b1_ref, w2_ref, b2_ref, out_ref):
    t = h_ref[...]
    a = jnp.dot(t, w1_ref[...], preferred_element_type=jnp.float32) + b1_ref[...]
    a = jnp.maximum(a, 0.0)
    out_ref[...] = jnp.dot(a, w2_ref[...], preferred_element_type=jnp.float32) + b2_ref[...]


def _ln_body(x_ref, p_ref, h_ref, g_ref, b_ref, out_ref):
    xseg = x_ref[0]
    xup = xseg + jnp.dot(p_ref[0], h_ref[0], preferred_element_type=jnp.float32)
    mu = jnp.mean(xup, axis=1, keepdims=True)
    var = jnp.mean((xup - mu) ** 2, axis=1, keepdims=True)
    out_ref[0] = (xup - mu) / jnp.sqrt(var + 1e-5) * g_ref[...] + b_ref[...]


def kernel(x, Wq, Wk, Wv, Wo, betas, W1, b1, W2, b2, ln_g, ln_b, Ws, bs):
    f32 = jnp.float32
    x4 = x.reshape(G, FULL, D)

    s4, mask4, p4, xsel4 = pl.pallas_call(
        _routing_body,
        grid=(G,),
        in_specs=[
            pl.BlockSpec((1, FULL, D), lambda i: (i, 0, 0)),
            pl.BlockSpec((D, 1), lambda i: (0, 0)),
            pl.BlockSpec(memory_space=pltpu.SMEM),
        ],
        out_specs=[
            pl.BlockSpec((1, FULL, 1), lambda i: (i, 0, 0)),
            pl.BlockSpec((1, _ROWS, _LANES), lambda i: (i, 0, 0)),
            pl.BlockSpec((1, FULL, SEG), lambda i: (i, 0, 0)),
            pl.BlockSpec((1, SEG, D), lambda i: (i, 0, 0)),
        ],
        out_shape=[
            jax.ShapeDtypeStruct((G, FULL, 1), f32),
            jax.ShapeDtypeStruct((G, _ROWS, _LANES), f32),
            jax.ShapeDtypeStruct((G, FULL, SEG), f32),
            jax.ShapeDtypeStruct((G, SEG, D), f32),
        ],
    )(x4, Ws, bs.reshape(1, 1))

    xsel = xsel4.reshape(B, NTOK, D)
    beta_r = betas.reshape(H, 1, DV)

    h_att = pl.pallas_call(
        _attn_body,
        grid=(B,),
        in_specs=[
            pl.BlockSpec((1, NTOK, D), lambda i: (i, 0, 0)),
            pl.BlockSpec((D, H * DK), lambda i: (0, 0)),
            pl.BlockSpec((D, H * DK), lambda i: (0, 0)),
            pl.BlockSpec((D, H * DV), lambda i: (0, 0)),
            pl.BlockSpec((H * DV, D), lambda i: (0, 0)),
            pl.BlockSpec((H, 1, DV), lambda i: (0, 0, 0)),
        ],
        out_specs=pl.BlockSpec((1, NTOK, D), lambda i: (i, 0, 0)),
        out_shape=jax.ShapeDtypeStruct((B, NTOK, D), f32),
    )(xsel, Wq, Wk, Wv, Wo, beta_r)

    tok = B * NTOK
    blk = 256
    h_mlp = pl.pallas_call(
        _mlp_body,
        grid=(tok // blk,),
        in_specs=[
            pl.BlockSpec((blk, D), lambda i: (i, 0)),
            pl.BlockSpec((D, DH), lambda i: (0, 0)),
            pl.BlockSpec((1, DH), lambda i: (0, 0)),
            pl.BlockSpec((DH, D), lambda i: (0, 0)),
            pl.BlockSpec((1, D), lambda i: (0, 0)),
        ],
        out_specs=pl.BlockSpec((blk, D), lambda i: (i, 0)),
        out_shape=jax.ShapeDtypeStruct((tok, D), f32),
    )(h_att.reshape(tok, D), W1, b1.reshape(1, DH), W2, b2.reshape(1, D))

    out4 = pl.pallas_call(
        _ln_body,
        grid=(G,),
        in_specs=[
            pl.BlockSpec((1, FULL, D), lambda i: (i, 0, 0)),
            pl.BlockSpec((1, FULL, SEG), lambda i: (i, 0, 0)),
            pl.BlockSpec((1, SEG, D), lambda i: (i, 0, 0)),
            pl.BlockSpec((1, D), lambda i: (0, 0)),
            pl.BlockSpec((1, D), lambda i: (0, 0)),
        ],
        out_specs=pl.BlockSpec((1, FULL, D), lambda i: (i, 0, 0)),
        out_shape=jax.ShapeDtypeStruct((G, FULL, D), f32),
    )(x4, p4, h_mlp.reshape(G, SEG, D), ln_g.reshape(1, D), ln_b.reshape(1, D))

    out = out4.reshape(B, S, D)
    return out, mask4.reshape(B * S, 1), s4.reshape(B * S, 1)


# speedup vs baseline: 1.6988x; 1.0047x over previous
import jax
import jax.numpy as jnp
from jax import lax
from jax.experimental import pallas as pl
from jax.experimental.pallas import tpu as pltpu

B = 2
S = 4096
D = 1024
DH = 4096
DK = 64
DV = 64
H = 16
FULL = 2048
SEG = 256
G = B * (S // FULL)
NTOK = SEG * (S // FULL)


def _elu1(t):
    return jnp.where(t > 0, t + 1.0, jnp.exp(t))


_ROWS = 16
_LANES = FULL // _ROWS


def _routing_body(x_ref, ws_ref, bs_ref, s_ref, mask_ref, sel_ref, xsel_ref):
    xseg = x_ref[0]
    ws = ws_ref[...]
    s = jnp.dot(xseg, ws, preferred_element_type=jnp.float32) + bs_ref[0, 0]
    s_ref[0] = s

    s2 = s.reshape(_ROWS, _LANES)
    u = lax.bitcast_convert_type(s2, jnp.uint32)
    u = jnp.where(u >= jnp.uint32(0x80000000), ~u, u | jnp.uint32(0x80000000))

    def bit_step(i, t):
        cand = t | lax.shift_left(jnp.uint32(1), jnp.uint32(31) - i.astype(jnp.uint32))
        cnt = jnp.sum((u >= cand).astype(jnp.int32))
        return jnp.where(cnt >= SEG, cand, t)

    thr = lax.fori_loop(0, 32, bit_step, jnp.uint32(0))

    gtf = (u > thr).astype(jnp.float32)
    eqf = (u == thr).astype(jnp.float32)
    need = jnp.float32(SEG) - jnp.sum(gtf)
    cu = lax.broadcasted_iota(jnp.int32, (_LANES, _LANES), 0)
    cv = lax.broadcasted_iota(jnp.int32, (_LANES, _LANES), 1)
    c128 = (cu < cv).astype(jnp.float32)
    ru = lax.broadcasted_iota(jnp.int32, (_ROWS, _ROWS), 0)
    rv = lax.broadcasted_iota(jnp.int32, (_ROWS, _ROWS), 1)
    r16 = (ru < rv).astype(jnp.float32)
    eprefix = jnp.dot(eqf, c128, preferred_element_type=jnp.float32)
    gprefix = jnp.dot(gtf, c128, preferred_element_type=jnp.float32)
    esum = jnp.sum(eqf, axis=1, keepdims=True)
    gsum = jnp.sum(gtf, axis=1, keepdims=True)
    eoff = lax.dot_general(r16, esum, (((0,), (0,)), ((), ())),
                           preferred_element_type=jnp.float32)
    goff = lax.dot_general(r16, gsum, (((0,), (0,)), ((), ())),
                           preferred_element_type=jnp.float32)
    erank = eprefix + eoff
    grank = gprefix + goff
    m2 = gtf + eqf * (erank < need).astype(jnp.float32)
    mask_ref[0] = m2
    rank = grank + jnp.minimum(erank, need)
    e_rows = (lax.broadcasted_iota(jnp.int32, (FULL, _ROWS), 0) // _LANES
              == lax.broadcasted_iota(jnp.int32, (FULL, _ROWS), 1)).astype(jnp.float32)
    lane_sel = (lax.broadcasted_iota(jnp.int32, (FULL, _LANES), 0) % _LANES
                == lax.broadcasted_iota(jnp.int32, (FULL, _LANES), 1))
    rank_rows = jnp.dot(e_rows, rank, preferred_element_type=jnp.float32)
    m_rows = jnp.dot(e_rows, m2, preferred_element_type=jnp.float32)
    rank_col = jnp.sum(jnp.where(lane_sel, rank_rows, 0.0), axis=1, keepdims=True)
    m_col = jnp.sum(jnp.where(lane_sel, m_rows, 0.0), axis=1, keepdims=True)
    cols = lax.broadcasted_iota(jnp.int32, (FULL, SEG), 1).astype(jnp.float32)
    p = m_col * (rank_col == cols).astype(jnp.float32)
    idx_col = lax.broadcasted_iota(jnp.int32, (FULL, 1), 0).astype(jnp.float32)
    sel_ref[0] = lax.dot_general(idx_col, p, (((0,), (0,)), ((), ())),
                                 preferred_element_type=jnp.float32)
    xsel_ref[0] = lax.dot_general(p, xseg, (((0,), (0,)), ((), ())),
                                  preferred_element_type=jnp.float32)


def _attn_body(xsel_ref, wq_ref, wk_ref, wv_ref, wo_ref, beta_ref, out_ref):
    xs = xsel_ref[0]
    q_all = jnp.dot(xs, wq_ref[...], preferred_element_type=jnp.float32)
    k_all = jnp.dot(xs, wk_ref[...], preferred_element_type=jnp.float32)
    v_all = jnp.dot(xs, wv_ref[...], preferred_element_type=jnp.float32)
    heads = []
    for h in range(H):
        q = q_all[:, h * DK:(h + 1) * DK]
        k = k_all[:, h * DK:(h + 1) * DK]
        v = v_all[:, h * DV:(h + 1) * DV]
        g = 1.0 / (1.0 + jnp.exp(-beta_ref[h]))
        mem = jnp.zeros((DK, DV), jnp.float32)
        zrow = jnp.ones((1, DK), jnp.float32) / DK
        outs = []
        for sgi in range(NTOK // SEG):
            qs = q[sgi * SEG:(sgi + 1) * SEG]
            ks = k[sgi * SEG:(sgi + 1) * SEG]
            vs = v[sgi * SEG:(sgi + 1) * SEG]
            sq = _elu1(qs)
            att_mem = jnp.dot(sq, mem, preferred_element_type=jnp.float32)
            att_mem = att_mem / lax.dot_general(sq, zrow, (((1,), (1,)), ((), ())),
                                                preferred_element_type=jnp.float32)
            scores = lax.dot_general(qs, ks, (((1,), (1,)), ((), ())),
                                     preferred_element_type=jnp.float32) / 8.0
            att_dot = jnp.dot(jax.nn.softmax(scores, axis=-1), vs,
                              preferred_element_type=jnp.float32)
            sk = _elu1(ks)
            mem = mem + lax.dot_general(sk, vs, (((0,), (0,)), ((), ())),
                                        preferred_element_type=jnp.float32)
            zrow = zrow + jnp.sum(sk, axis=0, keepdims=True)
            outs.append(g * att_mem + (1.0 - g) * att_dot)
        heads.append(jnp.concatenate(outs, axis=0))
    att_full = jnp.concatenate(heads, axis=1)
    out_ref[0] = jnp.dot(att_full, wo_ref[...], preferred_element_type=jnp.float32)


def _mlp_body(h_ref, w1_ref, b1_ref, w2_ref, b2_ref, out_ref):
    bf = jnp.bfloat16
    t = h_ref[...]
    a = jnp.dot(t.astype(bf), w1_ref[...].astype(bf),
                preferred_element_type=jnp.float32) + b1_ref[...]
    a = jnp.maximum(a, 0.0)
    out_ref[...] = jnp.dot(a.astype(bf), w2_ref[...].astype(bf),
                           preferred_element_type=jnp.float32) + b2_ref[...]


def _ln_body(x_ref, sel_ref, h_ref, g_ref, b_ref, out_ref):
    xseg = x_ref[0]
    rows = lax.broadcasted_iota(jnp.int32, (FULL, SEG), 0).astype(jnp.float32)
    p = (rows == sel_ref[0]).astype(jnp.float32)
    xup = xseg + jnp.dot(p, h_ref[0], preferred_element_type=jnp.float32)
    mu = jnp.mean(xup, axis=1, keepdims=True)
    var = jnp.mean((xup - mu) ** 2, axis=1, keepdims=True)
    out_ref[0] = (xup - mu) / jnp.sqrt(var + 1e-5) * g_ref[...] + b_ref[...]


def kernel(x, Wq, Wk, Wv, Wo, betas, W1, b1, W2, b2, ln_g, ln_b, Ws, bs):
    f32 = jnp.float32
    x4 = x.reshape(G, FULL, D)

    s4, mask4, sel4, xsel4 = pl.pallas_call(
        _routing_body,
        grid=(G,),
        in_specs=[
            pl.BlockSpec((1, FULL, D), lambda i: (i, 0, 0)),
            pl.BlockSpec((D, 1), lambda i: (0, 0)),
            pl.BlockSpec(memory_space=pltpu.SMEM),
        ],
        out_specs=[
            pl.BlockSpec((1, FULL, 1), lambda i: (i, 0, 0)),
            pl.BlockSpec((1, _ROWS, _LANES), lambda i: (i, 0, 0)),
            pl.BlockSpec((1, 1, SEG), lambda i: (i, 0, 0)),
            pl.BlockSpec((1, SEG, D), lambda i: (i, 0, 0)),
        ],
        out_shape=[
            jax.ShapeDtypeStruct((G, FULL, 1), f32),
            jax.ShapeDtypeStruct((G, _ROWS, _LANES), f32),
            jax.ShapeDtypeStruct((G, 1, SEG), f32),
            jax.ShapeDtypeStruct((G, SEG, D), f32),
        ],
    )(x4, Ws, bs.reshape(1, 1))

    xsel = xsel4.reshape(B, NTOK, D)
    beta_r = betas.reshape(H, 1, DV)

    h_att = pl.pallas_call(
        _attn_body,
        grid=(B,),
        in_specs=[
            pl.BlockSpec((1, NTOK, D), lambda i: (i, 0, 0)),
            pl.BlockSpec((D, H * DK), lambda i: (0, 0)),
            pl.BlockSpec((D, H * DK), lambda i: (0, 0)),
            pl.BlockSpec((D, H * DV), lambda i: (0, 0)),
            pl.BlockSpec((H * DV, D), lambda i: (0, 0)),
            pl.BlockSpec((H, 1, DV), lambda i: (0, 0, 0)),
        ],
        out_specs=pl.BlockSpec((1, NTOK, D), lambda i: (i, 0, 0)),
        out_shape=jax.ShapeDtypeStruct((B, NTOK, D), f32),
    )(xsel, Wq, Wk, Wv, Wo, beta_r)

    tok = B * NTOK
    blk = 256
    h_mlp = pl.pallas_call(
        _mlp_body,
        grid=(tok // blk,),
        in_specs=[
            pl.BlockSpec((blk, D), lambda i: (i, 0)),
            pl.BlockSpec((D, DH), lambda i: (0, 0)),
            pl.BlockSpec((1, DH), lambda i: (0, 0)),
            pl.BlockSpec((DH, D), lambda i: (0, 0)),
            pl.BlockSpec((1, D), lambda i: (0, 0)),
        ],
        out_specs=pl.BlockSpec((blk, D), lambda i: (i, 0)),
        out_shape=jax.ShapeDtypeStruct((tok, D), f32),
    )(h_att.reshape(tok, D), W1, b1.reshape(1, DH), W2, b2.reshape(1, D))

    out4 = pl.pallas_call(
        _ln_body,
        grid=(G,),
        in_specs=[
            pl.BlockSpec((1, FULL, D), lambda i: (i, 0, 0)),
            pl.BlockSpec((1, 1, SEG), lambda i: (i, 0, 0)),
            pl.BlockSpec((1, SEG, D), lambda i: (i, 0, 0)),
            pl.BlockSpec((1, D), lambda i: (0, 0)),
            pl.BlockSpec((1, D), lambda i: (0, 0)),
        ],
        out_specs=pl.BlockSpec((1, FULL, D), lambda i: (i, 0, 0)),
        out_shape=jax.ShapeDtypeStruct((G, FULL, D), f32),
    )(x4, sel4, h_mlp.reshape(G, SEG, D), ln_g.reshape(1, D), ln_b.reshape(1, D))

    out = out4.reshape(B, S, D)
    return out, mask4.reshape(B * S, 1), s4.reshape(B * S, 1)


# speedup vs baseline: 1.7129x; 1.0083x over previous
import jax
import jax.numpy as jnp
from jax import lax
from jax.experimental import pallas as pl
from jax.experimental.pallas import tpu as pltpu

B = 2
S = 4096
D = 1024
DH = 4096
DK = 64
DV = 64
H = 16
FULL = 2048
SEG = 256
G = B * (S // FULL)
NTOK = SEG * (S // FULL)


def _elu1(t):
    return jnp.where(t > 0, t + 1.0, jnp.exp(t))


_ROWS = 16
_LANES = FULL // _ROWS


def _routing_body(x_ref, ws_ref, bs_ref, s_ref, mask_ref, sel_ref, xsel_ref):
    xseg = x_ref[0]
    ws = ws_ref[...]
    s = jnp.dot(xseg, ws, preferred_element_type=jnp.float32) + bs_ref[0, 0]
    s_ref[0] = s

    s2 = s.reshape(_ROWS, _LANES)
    u = lax.bitcast_convert_type(s2, jnp.uint32)
    u = jnp.where(u >= jnp.uint32(0x80000000), ~u, u | jnp.uint32(0x80000000))

    def bit_step(i, t):
        cand = t | lax.shift_left(jnp.uint32(1), jnp.uint32(31) - i.astype(jnp.uint32))
        cnt = jnp.sum((u >= cand).astype(jnp.int32))
        return jnp.where(cnt >= SEG, cand, t)

    thr = lax.fori_loop(0, 32, bit_step, jnp.uint32(0))

    gtf = (u > thr).astype(jnp.float32)
    eqf = (u == thr).astype(jnp.float32)
    need = jnp.float32(SEG) - jnp.sum(gtf)
    cu = lax.broadcasted_iota(jnp.int32, (_LANES, _LANES), 0)
    cv = lax.broadcasted_iota(jnp.int32, (_LANES, _LANES), 1)
    c128 = (cu < cv).astype(jnp.float32)
    ru = lax.broadcasted_iota(jnp.int32, (_ROWS, _ROWS), 0)
    rv = lax.broadcasted_iota(jnp.int32, (_ROWS, _ROWS), 1)
    r16 = (ru < rv).astype(jnp.float32)
    eprefix = jnp.dot(eqf, c128, preferred_element_type=jnp.float32)
    gprefix = jnp.dot(gtf, c128, preferred_element_type=jnp.float32)
    esum = jnp.sum(eqf, axis=1, keepdims=True)
    gsum = jnp.sum(gtf, axis=1, keepdims=True)
    eoff = lax.dot_general(r16, esum, (((0,), (0,)), ((), ())),
                           preferred_element_type=jnp.float32)
    goff = lax.dot_general(r16, gsum, (((0,), (0,)), ((), ())),
                           preferred_element_type=jnp.float32)
    erank = eprefix + eoff
    grank = gprefix + goff
    m2 = gtf + eqf * (erank < need).astype(jnp.float32)
    mask_ref[0] = m2
    rank = grank + jnp.minimum(erank, need)
    e_rows = (lax.broadcasted_iota(jnp.int32, (FULL, _ROWS), 0) // _LANES
              == lax.broadcasted_iota(jnp.int32, (FULL, _ROWS), 1)).astype(jnp.float32)
    lane_sel = (lax.broadcasted_iota(jnp.int32, (FULL, _LANES), 0) % _LANES
                == lax.broadcasted_iota(jnp.int32, (FULL, _LANES), 1))
    rank_rows = jnp.dot(e_rows, rank, preferred_element_type=jnp.float32)
    m_rows = jnp.dot(e_rows, m2, preferred_element_type=jnp.float32)
    rank_col = jnp.sum(jnp.where(lane_sel, rank_rows, 0.0), axis=1, keepdims=True)
    m_col = jnp.sum(jnp.where(lane_sel, m_rows, 0.0), axis=1, keepdims=True)
    cols = lax.broadcasted_iota(jnp.int32, (FULL, SEG), 1).astype(jnp.float32)
    p = m_col * (rank_col == cols).astype(jnp.float32)
    idx_col = lax.broadcasted_iota(jnp.int32, (FULL, 1), 0).astype(jnp.float32)
    sel_ref[0] = jnp.sum(p * idx_col, axis=0, keepdims=True)
    xsel_ref[0] = lax.dot_general(p, xseg, (((0,), (0,)), ((), ())),
                                  preferred_element_type=jnp.float32)


def _attn_body(xsel_ref, wq_ref, wk_ref, wv_ref, wo_ref, beta_ref, out_ref):
    xs = xsel_ref[0]
    q_all = jnp.dot(xs, wq_ref[...], preferred_element_type=jnp.float32)
    k_all = jnp.dot(xs, wk_ref[...], preferred_element_type=jnp.float32)
    v_all = jnp.dot(xs, wv_ref[...], preferred_element_type=jnp.float32)
    heads = []
    for h in range(H):
        q = q_all[:, h * DK:(h + 1) * DK]
        k = k_all[:, h * DK:(h + 1) * DK]
        v = v_all[:, h * DV:(h + 1) * DV]
        g = 1.0 / (1.0 + jnp.exp(-beta_ref[h]))
        mem = jnp.zeros((DK, DV), jnp.float32)
        zrow = jnp.ones((1, DK), jnp.float32) / DK
        outs = []
        for sgi in range(NTOK // SEG):
            qs = q[sgi * SEG:(sgi + 1) * SEG]
            ks = k[sgi * SEG:(sgi + 1) * SEG]
            vs = v[sgi * SEG:(sgi + 1) * SEG]
            sq = _elu1(qs)
            att_mem = jnp.dot(sq, mem, preferred_element_type=jnp.float32)
            att_mem = att_mem / lax.dot_general(sq, zrow, (((1,), (1,)), ((), ())),
                                                preferred_element_type=jnp.float32)
            scores = lax.dot_general(qs, ks, (((1,), (1,)), ((), ())),
                                     preferred_element_type=jnp.float32) / 8.0
            att_dot = jnp.dot(jax.nn.softmax(scores, axis=-1), vs,
                              preferred_element_type=jnp.float32)
            sk = _elu1(ks)
            mem = mem + lax.dot_general(sk, vs, (((0,), (0,)), ((), ())),
                                        preferred_element_type=jnp.float32)
            zrow = zrow + jnp.sum(sk, axis=0, keepdims=True)
            outs.append(g * att_mem + (1.0 - g) * att_dot)
        heads.append(jnp.concatenate(outs, axis=0))
    att_full = jnp.concatenate(heads, axis=1)
    out_ref[0] = jnp.dot(att_full, wo_ref[...], preferred_element_type=jnp.float32)


def _mlp_body(h_ref, w1_ref, b1_ref, w2_ref, b2_ref, out_ref):
    bf = jnp.bfloat16
    t = h_ref[...]
    a = jnp.dot(t.astype(bf), w1_ref[...].astype(bf),
                preferred_element_type=jnp.float32) + b1_ref[...]
    a = jnp.maximum(a, 0.0)
    out_ref[...] = jnp.dot(a.astype(bf), w2_ref[...].astype(bf),
                           preferred_element_type=jnp.float32) + b2_ref[...]


def _ln_body(x_ref, sel_ref, h_ref, g_ref, b_ref, out_ref):
    xseg = x_ref[0]
    rows = lax.broadcasted_iota(jnp.int32, (FULL, SEG), 0).astype(jnp.float32)
    p = (rows == sel_ref[0]).astype(jnp.float32)
    xup = xseg + jnp.dot(p, h_ref[0], preferred_element_type=jnp.float32)
    mu = jnp.mean(xup, axis=1, keepdims=True)
    var = jnp.mean((xup - mu) ** 2, axis=1, keepdims=True)
    out_ref[0] = (xup - mu) / jnp.sqrt(var + 1e-5) * g_ref[...] + b_ref[...]


def kernel(x, Wq, Wk, Wv, Wo, betas, W1, b1, W2, b2, ln_g, ln_b, Ws, bs):
    f32 = jnp.float32
    x4 = x.reshape(G, FULL, D)

    s4, mask4, sel4, xsel4 = pl.pallas_call(
        _routing_body,
        grid=(G,),
        in_specs=[
            pl.BlockSpec((1, FULL, D), lambda i: (i, 0, 0)),
            pl.BlockSpec((D, 1), lambda i: (0, 0)),
            pl.BlockSpec(memory_space=pltpu.SMEM),
        ],
        out_specs=[
            pl.BlockSpec((1, FULL, 1), lambda i: (i, 0, 0)),
            pl.BlockSpec((1, _ROWS, _LANES), lambda i: (i, 0, 0)),
            pl.BlockSpec((1, 1, SEG), lambda i: (i, 0, 0)),
            pl.BlockSpec((1, SEG, D), lambda i: (i, 0, 0)),
        ],
        out_shape=[
            jax.ShapeDtypeStruct((G, FULL, 1), f32),
            jax.ShapeDtypeStruct((G, _ROWS, _LANES), f32),
            jax.ShapeDtypeStruct((G, 1, SEG), f32),
            jax.ShapeDtypeStruct((G, SEG, D), f32),
        ],
    )(x4, Ws, bs.reshape(1, 1))

    xsel = xsel4.reshape(B, NTOK, D)
    beta_r = betas.reshape(H, 1, DV)

    h_att = pl.pallas_call(
        _attn_body,
        grid=(B,),
        in_specs=[
            pl.BlockSpec((1, NTOK, D), lambda i: (i, 0, 0)),
            pl.BlockSpec((D, H * DK), lambda i: (0, 0)),
            pl.BlockSpec((D, H * DK), lambda i: (0, 0)),
            pl.BlockSpec((D, H * DV), lambda i: (0, 0)),
            pl.BlockSpec((H * DV, D), lambda i: (0, 0)),
            pl.BlockSpec((H, 1, DV), lambda i: (0, 0, 0)),
        ],
        out_specs=pl.BlockSpec((1, NTOK, D), lambda i: (i, 0, 0)),
        out_shape=jax.ShapeDtypeStruct((B, NTOK, D), f32),
    )(xsel, Wq, Wk, Wv, Wo, beta_r)

    tok = B * NTOK
    blk = 256
    h_mlp = pl.pallas_call(
        _mlp_body,
        grid=(tok // blk,),
        in_specs=[
            pl.BlockSpec((blk, D), lambda i: (i, 0)),
            pl.BlockSpec((D, DH), lambda i: (0, 0)),
            pl.BlockSpec((1, DH), lambda i: (0, 0)),
            pl.BlockSpec((DH, D), lambda i: (0, 0)),
            pl.BlockSpec((1, D), lambda i: (0, 0)),
        ],
        out_specs=pl.BlockSpec((blk, D), lambda i: (i, 0)),
        out_shape=jax.ShapeDtypeStruct((tok, D), f32),
    )(h_att.reshape(tok, D), W1, b1.reshape(1, DH), W2, b2.reshape(1, D))

    out4 = pl.pallas_call(
        _ln_body,
        grid=(G,),
        in_specs=[
            pl.BlockSpec((1, FULL, D), lambda i: (i, 0, 0)),
            pl.BlockSpec((1, 1, SEG), lambda i: (i, 0, 0)),
            pl.BlockSpec((1, SEG, D), lambda i: (i, 0, 0)),
            pl.BlockSpec((1, D), lambda i: (0, 0)),
            pl.BlockSpec((1, D), lambda i: (0, 0)),
        ],
        out_specs=pl.BlockSpec((1, FULL, D), lambda i: (i, 0, 0)),
        out_shape=jax.ShapeDtypeStruct((G, FULL, D), f32),
    )(x4, sel4, h_mlp.reshape(G, SEG, D), ln_g.reshape(1, D), ln_b.reshape(1, D))

    out = out4.reshape(B, S, D)
    return out, mask4.reshape(B * S, 1), s4.reshape(B * S, 1)
